# Initial kernel scaffold; baseline (speedup 1.0000x reference)
import jax
import jax.numpy as jnp
from jax.experimental import pallas as pl


def kernel(x, edge_index, edge_attr, params):
    raise NotImplementedError("write your pallas kernel here")


# speedup vs baseline: 4.9911x; 4.9911x over previous
import functools

import jax
import jax.numpy as jnp
from jax import lax
from jax.experimental import pallas as pl
from jax.experimental.pallas import tpu as pltpu
from jax.experimental.pallas import tpu_sc as plsc

F32 = jnp.float32
EPS = 1e-5

NW = 32
SB = 4000
BE = 2560
BN = 2560

_MESH = None


def _mesh():
    global _MESH
    if _MESH is None:
        _MESH = plsc.VectorSubcoreMesh(
            core_axis_name="c", subcore_axis_name="s", num_cores=2,
            num_subcores=16)
    return _MESH


_SC_PARAMS = pltpu.CompilerParams(needs_layout_passes=False)


def _wid():
    return lax.axis_index("s") * 2 + lax.axis_index("c")


def _worker_blocks(wid, nblk, fn):
    nb = (nblk - wid + NW - 1) // NW

    def body(k, _):
        fn((wid + k * NW) * SB)
        return 0

    lax.fori_loop(0, nb, body, 0)


def _gather_block(table_v, idx_v, out_v):
    def body(i, _):
        base = i * 80
        for u in range(5):
            o = base + u * 16
            idx = idx_v[pl.ds(o, 16)]
            out_v[pl.ds(o, 16)] = plsc.load_gather(table_v, [idx])
        return 0

    lax.fori_loop(0, SB // 80, body, 0)


def _scatter_block(acc_v, idx_v, val_v):
    def body(i, _):
        base = i * 80
        for u in range(5):
            o = base + u * 16
            idx = idx_v[pl.ds(o, 16)]
            v = val_v[pl.ds(o, 16)]
            plsc.addupdate_scatter(acc_v, [idx], v)
        return 0

    lax.fori_loop(0, SB // 80, body, 0)


def _fill(ref, n, value):
    vec = jnp.full((16,), value, F32)

    def body(i, _):
        base = i * 160
        for u in range(10):
            ref[pl.ds(base + u * 16, 16)] = vec
        return 0

    lax.fori_loop(0, n // 160, body, 0)


def _sc_gather_x(xT, tgt):
    n = xT.shape[1]
    e = tgt.shape[0]
    nblk = e // SB

    @functools.partial(
        pl.kernel,
        out_type=jax.ShapeDtypeStruct((3 * e,), F32),
        mesh=_mesh(),
        scratch_types=[
            pltpu.VMEM((n,), F32),
            pltpu.VMEM((SB,), jnp.int32),
            pltpu.VMEM((SB,), F32),
        ],
        compiler_params=_SC_PARAMS,
    )
    def k(xT_h, tgt_h, out_h, table_v, idx_v, out_v):
        wid = _wid()
        for j in range(3):
            pltpu.sync_copy(xT_h.at[pl.ds(j * n, n)], table_v)

            def do(base, j=j):
                pltpu.sync_copy(tgt_h.at[pl.ds(base, SB)], idx_v)
                _gather_block(table_v, idx_v, out_v)
                pltpu.sync_copy(out_v, out_h.at[pl.ds(j * e + base, SB)])

            _worker_blocks(wid, nblk, do)

    return k(xT.reshape(-1), tgt).reshape(3, e)


def _sc_gather_xe(xeT, tgt, src):
    n = xeT.shape[1]
    e = tgt.shape[0]
    nblk = e // SB

    @functools.partial(
        pl.kernel,
        out_type=jax.ShapeDtypeStruct((4 * e,), F32),
        mesh=_mesh(),
        scratch_types=[
            pltpu.VMEM((n,), F32),
            pltpu.VMEM((SB,), jnp.int32),
            pltpu.VMEM((SB,), F32),
        ],
        compiler_params=_SC_PARAMS,
    )
    def k(xeT_h, tgt_h, src_h, out_h, table_v, idx_v, out_v):
        wid = _wid()
        for j in range(2):
            pltpu.sync_copy(xeT_h.at[pl.ds(j * n, n)], table_v)
            for r, idx_h in ((j, tgt_h), (2 + j, src_h)):

                def do(base, r=r, idx_h=idx_h):
                    pltpu.sync_copy(idx_h.at[pl.ds(base, SB)], idx_v)
                    _gather_block(table_v, idx_v, out_v)
                    pltpu.sync_copy(out_v, out_h.at[pl.ds(r * e + base, SB)])

                _worker_blocks(wid, nblk, do)

    return k(xeT.reshape(-1), tgt, src).reshape(4, e)


def _sc_scatter4(tgt, src, msgT, n):
    e = tgt.shape[0]
    nblk = e // SB

    @functools.partial(
        pl.kernel,
        out_type=jax.ShapeDtypeStruct((4 * NW * n,), F32),
        mesh=_mesh(),
        scratch_types=[
            pltpu.VMEM((n,), F32),
            pltpu.VMEM((SB,), jnp.int32),
            pltpu.VMEM((SB,), F32),
        ],
        compiler_params=_SC_PARAMS,
    )
    def k(tgt_h, src_h, msg_h, out_h, acc_v, idx_v, val_v):
        wid = _wid()
        for col in range(4):
            _fill(acc_v, n, 0.0)
            if col >= 2:
                _fill(val_v, SB, 1.0)
            idx_h = src_h if col == 3 else tgt_h

            def do(base, col=col, idx_h=idx_h):
                pltpu.sync_copy(idx_h.at[pl.ds(base, SB)], idx_v)
                if col < 2:
                    pltpu.sync_copy(msg_h.at[pl.ds(col * e + base, SB)],
                                    val_v)
                _scatter_block(acc_v, idx_v, val_v)

            _worker_blocks(wid, nblk, do)
            pltpu.sync_copy(acc_v, out_h.at[pl.ds((col * NW + wid) * n, n)])

    return k(tgt, src, msgT.reshape(-1)).reshape(4, NW, n)


def _sc_scatter3(tgt, msgT, n):
    e = tgt.shape[0]
    nblk = e // SB

    @functools.partial(
        pl.kernel,
        out_type=jax.ShapeDtypeStruct((3 * NW * n,), F32),
        mesh=_mesh(),
        scratch_types=[
            pltpu.VMEM((n,), F32),
            pltpu.VMEM((SB,), jnp.int32),
            pltpu.VMEM((SB,), F32),
        ],
        compiler_params=_SC_PARAMS,
    )
    def k(tgt_h, msg_h, out_h, acc_v, idx_v, val_v):
        wid = _wid()
        for col in range(3):
            _fill(acc_v, n, 0.0)

            def do(base, col=col):
                pltpu.sync_copy(tgt_h.at[pl.ds(base, SB)], idx_v)
                pltpu.sync_copy(msg_h.at[pl.ds(col * e + base, SB)], val_v)
                _scatter_block(acc_v, idx_v, val_v)

            _worker_blocks(wid, nblk, do)
            pltpu.sync_copy(acc_v, out_h.at[pl.ds((col * NW + wid) * n, n)])

    return k(tgt, msgT.reshape(-1)).reshape(3, NW, n)


def _dgf(a, x):
    return lax.dot_general(a, x, (((0,), (0,)), ((), ())),
                           preferred_element_type=F32)


def _params_tc(dimsem="arbitrary"):
    return pltpu.CompilerParams(dimension_semantics=(dimsem,))


def _full(shape):
    return pl.BlockSpec(shape, lambda i: (0,) * len(shape))


def _colstats(xT):
    f, n = xT.shape

    def body(x_ref, st_ref):
        i = pl.program_id(0)
        xb = x_ref[...]
        s = jnp.sum(xb, axis=1, keepdims=True)
        q = jnp.sum(xb * xb, axis=1, keepdims=True)
        blk = jnp.concatenate([s, q], axis=1)

        @pl.when(i == 0)
        def _():
            st_ref[...] = jnp.zeros_like(st_ref)

        st_ref[...] += blk

    return pl.pallas_call(
        body,
        grid=(n // BN,),
        in_specs=[pl.BlockSpec((f, BN), lambda i: (0, i))],
        out_specs=_full((f, 2)),
        out_shape=jax.ShapeDtypeStruct((f, 2), F32),
        compiler_params=_params_tc(),
    )(xT)


def _enc_mlp(xg, ea, Wx, Wea, b0, W2, b2, W3, b3, W4, b4):
    e = xg.shape[1]

    def body(xg_ref, ea_ref, Wx_r, Wea_r, b0_r, W2_r, b2_r, W3_r, b3_r,
             W4_r, b4_r, out_ref):
        h = _dgf(Wx_r[...], xg_ref[...])
        h = h + lax.dot_general(Wea_r[...], ea_ref[...],
                                (((0,), (1,)), ((), ())),
                                preferred_element_type=F32)
        h = jnp.maximum(h + b0_r[...], 0.0)
        h = jnp.maximum(_dgf(W2_r[...], h) + b2_r[...], 0.0)
        h = jnp.maximum(_dgf(W3_r[...], h) + b3_r[...], 0.0)
        h = jnp.maximum(_dgf(W4_r[...], h) + b4_r[...], 0.0)
        out_ref[...] = h

    return pl.pallas_call(
        body,
        grid=(e // BE,),
        in_specs=[
            pl.BlockSpec((3, BE), lambda i: (0, i)),
            pl.BlockSpec((BE, 4), lambda i: (i, 0)),
            _full(Wx.shape), _full(Wea.shape), _full(b0.shape),
            _full(W2.shape), _full(b2.shape), _full(W3.shape),
            _full(b3.shape), _full(W4.shape), _full(b4.shape),
        ],
        out_specs=pl.BlockSpec((2, BE), lambda i: (0, i)),
        out_shape=jax.ShapeDtypeStruct((2, e), F32),
        compiler_params=_params_tc(),
    )(xg, ea, Wx, Wea, b0, W2, b2, W3, b3, W4, b4)


def _node_finalize_a(part):
    n = part.shape[2]

    def body(p_ref, raw_ref, cnt_ref, st_ref):
        i = pl.program_id(0)
        p = jnp.sum(p_ref[...], axis=1)
        cnt_t = p[2:3]
        safe = jnp.maximum(cnt_t, 1.0)
        raw = p[0:2] / safe
        raw_ref[...] = raw
        cnt_ref[...] = p[2:4]
        s = jnp.sum(raw, axis=1, keepdims=True)
        q = jnp.sum(raw * raw, axis=1, keepdims=True)
        blk = jnp.concatenate([s, q], axis=1)

        @pl.when(i == 0)
        def _():
            st_ref[...] = jnp.zeros_like(st_ref)

        st_ref[...] += blk

    return pl.pallas_call(
        body,
        grid=(n // BN,),
        in_specs=[pl.BlockSpec((4, NW, BN), lambda i: (0, 0, i))],
        out_specs=[
            pl.BlockSpec((2, BN), lambda i: (0, i)),
            pl.BlockSpec((2, BN), lambda i: (0, i)),
            _full((2, 2)),
        ],
        out_shape=[
            jax.ShapeDtypeStruct((2, n), F32),
            jax.ShapeDtypeStruct((2, n), F32),
            jax.ShapeDtypeStruct((2, 2), F32),
        ],
        compiler_params=_params_tc(),
    )(part)


def _node_finalize_b(raw, cnt2, a2c2):
    n = raw.shape[1]

    def body(raw_ref, cnt_ref, a_ref, xe_ref, ws_ref):
        i = pl.program_id(0)
        a2 = a_ref[:, 0:1]
        c2 = a_ref[:, 1:2]
        xe = raw_ref[...] * a2 + c2
        xe_ref[...] = xe
        ct = cnt_ref[0:1]
        cs = cnt_ref[1:2]
        xe2 = xe * xe
        blk = jnp.concatenate(
            [jnp.sum(cs * xe, axis=1, keepdims=True),
             jnp.sum(cs * xe2, axis=1, keepdims=True),
             jnp.sum(ct * xe, axis=1, keepdims=True),
             jnp.sum(ct * xe2, axis=1, keepdims=True)], axis=1)

        @pl.when(i == 0)
        def _():
            ws_ref[...] = jnp.zeros_like(ws_ref)

        ws_ref[...] += blk

    return pl.pallas_call(
        body,
        grid=(n // BN,),
        in_specs=[
            pl.BlockSpec((2, BN), lambda i: (0, i)),
            pl.BlockSpec((2, BN), lambda i: (0, i)),
            _full((2, 2)),
        ],
        out_specs=[
            pl.BlockSpec((2, BN), lambda i: (0, i)),
            _full((2, 4)),
        ],
        out_shape=[
            jax.ShapeDtypeStruct((2, n), F32),
            jax.ShapeDtypeStruct((2, 4), F32),
        ],
        compiler_params=_params_tc(),
    )(raw, cnt2, a2c2)


def _dec_ep1(xg2, Wd1, bd1, Wd2, bd2, Wd3, bd3, Wd4, bd4, W1s, W1t, b1f):
    e = xg2.shape[1]

    def body(g_ref, Wd1_r, bd1_r, Wd2_r, bd2_r, Wd3_r, bd3_r, Wd4_r, bd4_r,
             W1s_r, W1t_r, b1f_r, msg_ref, hs_ref):
        i = pl.program_id(0)
        g = g_ref[...]
        xi = g[0:2]
        xj = g[2:4]
        d = xj - xi
        W = Wd1_r[...]
        h = jnp.maximum(_dgf(W[0:2], xi) + _dgf(W[2:4], d) + bd1_r[...], 0.0)
        h = jnp.maximum(_dgf(Wd2_r[...], h) + bd2_r[...], 0.0)
        h = jnp.maximum(_dgf(Wd3_r[...], h) + bd3_r[...], 0.0)
        msg_ref[...] = _dgf(Wd4_r[...], h) + bd4_r[...]

        he = jnp.maximum(
            _dgf(W1s_r[...], xj) + _dgf(W1t_r[...], xi) + b1f_r[...], 0.0)
        s = jnp.sum(he, axis=1, keepdims=True)
        q = jnp.sum(he * he, axis=1, keepdims=True)
        blk = jnp.concatenate([s, q], axis=1)

        @pl.when(i == 0)
        def _():
            hs_ref[...] = jnp.zeros_like(hs_ref)

        hs_ref[...] += blk

    return pl.pallas_call(
        body,
        grid=(e // BE,),
        in_specs=[
            pl.BlockSpec((4, BE), lambda i: (0, i)),
            _full(Wd1.shape), _full(bd1.shape), _full(Wd2.shape),
            _full(bd2.shape), _full(Wd3.shape), _full(bd3.shape),
            _full(Wd4.shape), _full(bd4.shape), _full(W1s.shape),
            _full(W1t.shape), _full(b1f.shape),
        ],
        out_specs=[
            pl.BlockSpec((3, BE), lambda i: (0, i)),
            _full((32, 2)),
        ],
        out_shape=[
            jax.ShapeDtypeStruct((3, e), F32),
            jax.ShapeDtypeStruct((32, 2), F32),
        ],
        compiler_params=_params_tc(),
    )(xg2, Wd1, bd1, Wd2, bd2, Wd3, bd3, Wd4, bd4, W1s, W1t, b1f)


def _ep2_stats(xg2, W1s, W1t, b1f, a4, c4, W2, b2):
    e = xg2.shape[1]

    def body(g_ref, W1s_r, W1t_r, b1f_r, a4_r, c4_r, W2_r, b2_r, rs_ref):
        i = pl.program_id(0)
        g = g_ref[...]
        xi = g[0:2]
        xj = g[2:4]
        he = jnp.maximum(
            _dgf(W1s_r[...], xj) + _dgf(W1t_r[...], xi) + b1f_r[...], 0.0)
        hb = he * a4_r[...] + c4_r[...]
        hr = jnp.maximum(_dgf(W2_r[...], hb) + b2_r[...], 0.0)
        s = jnp.sum(hr, axis=1, keepdims=True)
        q = jnp.sum(hr * hr, axis=1, keepdims=True)
        blk = jnp.concatenate([s, q], axis=1)

        @pl.when(i == 0)
        def _():
            rs_ref[...] = jnp.zeros_like(rs_ref)

        rs_ref[...] += blk

    return pl.pallas_call(
        body,
        grid=(e // BE,),
        in_specs=[
            pl.BlockSpec((4, BE), lambda i: (0, i)),
            _full(W1s.shape), _full(W1t.shape), _full(b1f.shape),
            _full(a4.shape), _full(c4.shape), _full(W2.shape),
            _full(b2.shape),
        ],
        out_specs=_full((32, 2)),
        out_shape=jax.ShapeDtypeStruct((32, 2), F32),
        compiler_params=_params_tc(),
    )(xg2, W1s, W1t, b1f, a4, c4, W2, b2)


def _ep_final(xg2, W1s, W1t, b1f, a4, c4, W2, b2, a5, c5, W4, b4):
    e = xg2.shape[1]

    def body(g_ref, W1s_r, W1t_r, b1f_r, a4_r, c4_r, W2_r, b2_r, a5_r, c5_r,
             W4_r, b4_r, out_ref):
        g = g_ref[...]
        xi = g[0:2]
        xj = g[2:4]
        he = jnp.maximum(
            _dgf(W1s_r[...], xj) + _dgf(W1t_r[...], xi) + b1f_r[...], 0.0)
        hb = he * a4_r[...] + c4_r[...]
        hr = jnp.maximum(_dgf(W2_r[...], hb) + b2_r[...], 0.0)
        hh = hb + hr * a5_r[...] + c5_r[...]
        out_ref[...] = lax.dot_general(
            hh, W4_r[...], (((0,), (0,)), ((), ())),
            preferred_element_type=F32) + b4_r[...]

    return pl.pallas_call(
        body,
        grid=(e // BE,),
        in_specs=[
            pl.BlockSpec((4, BE), lambda i: (0, i)),
            _full(W1s.shape), _full(W1t.shape), _full(b1f.shape),
            _full(a4.shape), _full(c4.shape), _full(W2.shape),
            _full(b2.shape), _full(a5.shape), _full(c5.shape),
            _full(W4.shape), _full(b4.shape),
        ],
        out_specs=pl.BlockSpec((BE, 4), lambda i: (i, 0)),
        out_shape=jax.ShapeDtypeStruct((e, 4), F32),
        compiler_params=_params_tc(),
    )(xg2, W1s, W1t, b1f, a4, c4, W2, b2, a5, c5, W4, b4)


def _recon_finalize(part2, cnt2):
    n = part2.shape[2]

    def body(p_ref, cnt_ref, rec_ref):
        p = jnp.sum(p_ref[...], axis=1)
        safe = jnp.maximum(cnt_ref[0:1], 1.0)
        rec_ref[...] = p / safe

    return pl.pallas_call(
        body,
        grid=(n // BN,),
        in_specs=[
            pl.BlockSpec((3, NW, BN), lambda i: (0, 0, i)),
            pl.BlockSpec((2, BN), lambda i: (0, i)),
        ],
        out_specs=pl.BlockSpec((3, BN), lambda i: (0, i)),
        out_shape=jax.ShapeDtypeStruct((3, n), F32),
        compiler_params=_params_tc(),
    )(part2, cnt2)


def _affine(g, b, s, q, count):
    m = s / count
    v = q / count - m * m
    a = g / jnp.sqrt(v + EPS)
    return a, b - m * a


def kernel(x, edge_index, edge_attr, params):
    n = x.shape[0]
    e = edge_index.shape[1]
    npad = -(-n // BN) * BN
    src = edge_index[0].astype(jnp.int32)
    tgt = edge_index[1].astype(jnp.int32)
    xT = jnp.pad(x.T, ((0, 0), (0, npad - n)))

    st1 = _colstats(xT)
    a1, c1 = _affine(params['bn1g'], params['bn1b'], st1[:, 0], st1[:, 1], n)
    We0, be0 = params['enc'][0]
    Wx = We0[:3] * a1[:, None]
    Wea = We0[3:]
    b0 = (be0 + c1 @ We0[:3]).reshape(32, 1)

    xg = _sc_gather_x(xT, tgt)
    (W2e, b2e), (W3e, b3e), (W4e, b4e) = params['enc'][1:]
    msgT = _enc_mlp(xg, edge_attr, Wx, Wea, b0, W2e, b2e.reshape(-1, 1),
                    W3e, b3e.reshape(-1, 1), W4e, b4e.reshape(-1, 1))

    part = _sc_scatter4(tgt, src, msgT, npad)
    raw, cnt2, rst = _node_finalize_a(part)

    a2, c2 = _affine(params['bn2g'], params['bn2b'], rst[:, 0], rst[:, 1], n)
    a2c2 = jnp.stack([a2, c2], axis=1)
    xeT, ws = _node_finalize_b(raw, cnt2, a2c2)

    S_s, Q_s, S_t, Q_t = ws[:, 0], ws[:, 1], ws[:, 2], ws[:, 3]
    a3, c3 = _affine(params['epbn1g'], params['epbn1b'],
                     jnp.concatenate([S_s, S_t]),
                     jnp.concatenate([Q_s, Q_t]), e)
    W1, b1 = params['ep_fc1']
    W1s = W1[:2] * a3[:2, None]
    W1t = W1[2:] * a3[2:, None]
    b1f = (b1 + c3 @ W1).reshape(32, 1)

    xg2 = _sc_gather_xe(xeT, tgt, src)
    (Wd1, bd1), (Wd2, bd2), (Wd3, bd3), (Wd4, bd4) = params['dec']
    msg2T, hs = _dec_ep1(
        xg2, Wd1, bd1.reshape(-1, 1), Wd2, bd2.reshape(-1, 1),
        Wd3, bd3.reshape(-1, 1), Wd4, bd4.reshape(-1, 1), W1s, W1t, b1f)

    part2 = _sc_scatter3(tgt, msg2T, npad)
    recT = _recon_finalize(part2, cnt2)
    x_recon = recT[:, :n].T

    a4, c4 = _affine(params['epbn2g'], params['epbn2b'], hs[:, 0], hs[:, 1], e)
    a4 = a4.reshape(32, 1)
    c4 = c4.reshape(32, 1)
    W2, b2 = params['ep_fc2']
    rs = _ep2_stats(xg2, W1s, W1t, b1f, a4, c4, W2, b2.reshape(-1, 1))

    a5, c5 = _affine(params['epbn3g'], params['epbn3b'], rs[:, 0], rs[:, 1], e)
    W4, b4 = params['ep_fc4']
    pred = _ep_final(xg2, W1s, W1t, b1f, a4, c4, W2, b2.reshape(-1, 1),
                     a5.reshape(32, 1), c5.reshape(32, 1), W4,
                     b4.reshape(1, -1))
    return (x_recon, pred)


# speedup vs baseline: 6.3094x; 1.2641x over previous
import functools

import jax
import jax.numpy as jnp
from jax import lax
from jax.experimental import pallas as pl
from jax.experimental.pallas import tpu as pltpu
from jax.experimental.pallas import tpu_sc as plsc

F32 = jnp.float32
EPS = 1e-5

NW = 32
SB = 10000
BE = 6400
BN = 2560

_MESH = None


def _mesh():
    global _MESH
    if _MESH is None:
        _MESH = plsc.VectorSubcoreMesh(
            core_axis_name="c", subcore_axis_name="s", num_cores=2,
            num_subcores=16)
    return _MESH


_SC_PARAMS = pltpu.CompilerParams(needs_layout_passes=False)


def _wid():
    return lax.axis_index("s") * 2 + lax.axis_index("c")


def _worker_blocks(wid, nblk, fn):
    nb = nblk // NW

    def body(k, _):
        fn((wid * nb + k) * SB)
        return 0

    lax.fori_loop(0, nb, body, 0)


def _gather_block(table_v, idx_v, out_v):
    def body(i, _):
        base = i * 80
        for u in range(5):
            o = base + u * 16
            idx = idx_v[pl.ds(o, 16)]
            out_v[pl.ds(o, 16)] = plsc.load_gather(table_v, [idx])
        return 0

    lax.fori_loop(0, SB // 80, body, 0)


def _scatter_block(acc_v, idx_v, val_v):
    def body(i, _):
        base = i * 80
        for u in range(5):
            o = base + u * 16
            idx = idx_v[pl.ds(o, 16)]
            v = val_v[pl.ds(o, 16)]
            plsc.addupdate_scatter(acc_v, [idx], v)
        return 0

    lax.fori_loop(0, SB // 80, body, 0)


def _fill(ref, n, value):
    vec = jnp.full((16,), value, F32)

    def body(i, _):
        base = i * 80
        for u in range(5):
            ref[pl.ds(base + u * 16, 16)] = vec
        return 0

    lax.fori_loop(0, n // 80, body, 0)


def _sc_gather_x(xT, tgt):
    n = xT.shape[1]
    e = tgt.shape[0]
    nblk = e // SB

    @functools.partial(
        pl.kernel,
        out_type=jax.ShapeDtypeStruct((3 * e,), F32),
        mesh=_mesh(),
        scratch_types=[
            pltpu.VMEM((n,), F32),
            pltpu.VMEM((SB,), jnp.int32),
            pltpu.VMEM((SB,), F32),
        ],
        compiler_params=_SC_PARAMS,
    )
    def k(xT_h, tgt_h, out_h, table_v, idx_v, out_v):
        wid = _wid()
        for j in range(3):
            pltpu.sync_copy(xT_h.at[pl.ds(j * n, n)], table_v)

            def do(base, j=j):
                pltpu.sync_copy(tgt_h.at[pl.ds(base, SB)], idx_v)
                _gather_block(table_v, idx_v, out_v)
                pltpu.sync_copy(out_v, out_h.at[pl.ds(j * e + base, SB)])

            _worker_blocks(wid, nblk, do)

    return k(xT.reshape(-1), tgt).reshape(3, e)


def _sc_gather_xe(xeT, tgt, src):
    n = xeT.shape[1]
    e = tgt.shape[0]
    nblk = e // SB

    @functools.partial(
        pl.kernel,
        out_type=jax.ShapeDtypeStruct((4 * e,), F32),
        mesh=_mesh(),
        scratch_types=[
            pltpu.VMEM((n,), F32),
            pltpu.VMEM((SB,), jnp.int32),
            pltpu.VMEM((SB,), F32),
        ],
        compiler_params=_SC_PARAMS,
    )
    def k(xeT_h, tgt_h, src_h, out_h, table_v, idx_v, out_v):
        wid = _wid()
        for j in range(2):
            pltpu.sync_copy(xeT_h.at[pl.ds(j * n, n)], table_v)
            for r, idx_h in ((j, tgt_h), (2 + j, src_h)):

                def do(base, r=r, idx_h=idx_h):
                    pltpu.sync_copy(idx_h.at[pl.ds(base, SB)], idx_v)
                    _gather_block(table_v, idx_v, out_v)
                    pltpu.sync_copy(out_v, out_h.at[pl.ds(r * e + base, SB)])

                _worker_blocks(wid, nblk, do)

    return k(xeT.reshape(-1), tgt, src).reshape(4, e)


def _sc_scatter4(tgt, src, msgT, n):
    e = tgt.shape[0]
    nblk = e // SB

    @functools.partial(
        pl.kernel,
        out_type=jax.ShapeDtypeStruct((4 * NW * n,), F32),
        mesh=_mesh(),
        scratch_types=[
            pltpu.VMEM((n,), F32),
            pltpu.VMEM((SB,), jnp.int32),
            pltpu.VMEM((SB,), F32),
        ],
        compiler_params=_SC_PARAMS,
    )
    def k(tgt_h, src_h, msg_h, out_h, acc_v, idx_v, val_v):
        wid = _wid()
        for col in range(4):
            _fill(acc_v, n, 0.0)
            if col >= 2:
                _fill(val_v, SB, 1.0)
            idx_h = src_h if col == 3 else tgt_h

            def do(base, col=col, idx_h=idx_h):
                pltpu.sync_copy(idx_h.at[pl.ds(base, SB)], idx_v)
                if col < 2:
                    pltpu.sync_copy(msg_h.at[pl.ds(col * e + base, SB)],
                                    val_v)
                _scatter_block(acc_v, idx_v, val_v)

            _worker_blocks(wid, nblk, do)
            pltpu.sync_copy(acc_v, out_h.at[pl.ds((col * NW + wid) * n, n)])

    return k(tgt, src, msgT.reshape(-1)).reshape(4, NW, n)


def _sc_scatter3(tgt, msgT, n):
    e = tgt.shape[0]
    nblk = e // SB

    @functools.partial(
        pl.kernel,
        out_type=jax.ShapeDtypeStruct((3 * NW * n,), F32),
        mesh=_mesh(),
        scratch_types=[
            pltpu.VMEM((n,), F32),
            pltpu.VMEM((SB,), jnp.int32),
            pltpu.VMEM((SB,), F32),
        ],
        compiler_params=_SC_PARAMS,
    )
    def k(tgt_h, msg_h, out_h, acc_v, idx_v, val_v):
        wid = _wid()
        for col in range(3):
            _fill(acc_v, n, 0.0)

            def do(base, col=col):
                pltpu.sync_copy(tgt_h.at[pl.ds(base, SB)], idx_v)
                pltpu.sync_copy(msg_h.at[pl.ds(col * e + base, SB)], val_v)
                _scatter_block(acc_v, idx_v, val_v)

            _worker_blocks(wid, nblk, do)
            pltpu.sync_copy(acc_v, out_h.at[pl.ds((col * NW + wid) * n, n)])

    return k(tgt, msgT.reshape(-1)).reshape(3, NW, n)


def _dgf(a, x):
    return lax.dot_general(a, x, (((0,), (0,)), ((), ())),
                           preferred_element_type=F32)


def _params_tc(dimsem="arbitrary"):
    return pltpu.CompilerParams(dimension_semantics=(dimsem,))


def _full(shape):
    return pl.BlockSpec(shape, lambda i: (0,) * len(shape))


def _colstats(xT):
    f, n = xT.shape

    def body(x_ref, st_ref):
        i = pl.program_id(0)
        xb = x_ref[...]
        s = jnp.sum(xb, axis=1, keepdims=True)
        q = jnp.sum(xb * xb, axis=1, keepdims=True)
        blk = jnp.concatenate([s, q], axis=1)

        @pl.when(i == 0)
        def _():
            st_ref[...] = jnp.zeros_like(st_ref)

        st_ref[...] += blk

    return pl.pallas_call(
        body,
        grid=(n // BN,),
        in_specs=[pl.BlockSpec((f, BN), lambda i: (0, i))],
        out_specs=_full((f, 2)),
        out_shape=jax.ShapeDtypeStruct((f, 2), F32),
        compiler_params=_params_tc(),
    )(xT)


def _enc_mlp(xg, ea, Wx, Wea, b0, W2, b2, W3, b3, W4, b4):
    e = xg.shape[1]

    def body(xg_ref, ea_ref, Wx_r, Wea_r, b0_r, W2_r, b2_r, W3_r, b3_r,
             W4_r, b4_r, out_ref):
        h = _dgf(Wx_r[...], xg_ref[...])
        h = h + lax.dot_general(Wea_r[...], ea_ref[...],
                                (((0,), (1,)), ((), ())),
                                preferred_element_type=F32)
        h = jnp.maximum(h + b0_r[...], 0.0)
        h = jnp.maximum(_dgf(W2_r[...], h) + b2_r[...], 0.0)
        h = jnp.maximum(_dgf(W3_r[...], h) + b3_r[...], 0.0)
        h = jnp.maximum(_dgf(W4_r[...], h) + b4_r[...], 0.0)
        out_ref[...] = h

    return pl.pallas_call(
        body,
        grid=(e // BE,),
        in_specs=[
            pl.BlockSpec((3, BE), lambda i: (0, i)),
            pl.BlockSpec((BE, 4), lambda i: (i, 0)),
            _full(Wx.shape), _full(Wea.shape), _full(b0.shape),
            _full(W2.shape), _full(b2.shape), _full(W3.shape),
            _full(b3.shape), _full(W4.shape), _full(b4.shape),
        ],
        out_specs=pl.BlockSpec((2, BE), lambda i: (0, i)),
        out_shape=jax.ShapeDtypeStruct((2, e), F32),
        compiler_params=_params_tc(),
    )(xg, ea, Wx, Wea, b0, W2, b2, W3, b3, W4, b4)


def _node_finalize_a(part):
    n = part.shape[2]

    def body(p_ref, raw_ref, cnt_ref, st_ref):
        i = pl.program_id(0)
        p = jnp.sum(p_ref[...], axis=1)
        cnt_t = p[2:3]
        safe = jnp.maximum(cnt_t, 1.0)
        raw = p[0:2] / safe
        raw_ref[...] = raw
        cnt_ref[...] = p[2:4]
        s = jnp.sum(raw, axis=1, keepdims=True)
        q = jnp.sum(raw * raw, axis=1, keepdims=True)
        blk = jnp.concatenate([s, q], axis=1)

        @pl.when(i == 0)
        def _():
            st_ref[...] = jnp.zeros_like(st_ref)

        st_ref[...] += blk

    return pl.pallas_call(
        body,
        grid=(n // BN,),
        in_specs=[pl.BlockSpec((4, NW, BN), lambda i: (0, 0, i))],
        out_specs=[
            pl.BlockSpec((2, BN), lambda i: (0, i)),
            pl.BlockSpec((2, BN), lambda i: (0, i)),
            _full((2, 2)),
        ],
        out_shape=[
            jax.ShapeDtypeStruct((2, n), F32),
            jax.ShapeDtypeStruct((2, n), F32),
            jax.ShapeDtypeStruct((2, 2), F32),
        ],
        compiler_params=_params_tc(),
    )(part)


def _node_finalize_b(raw, cnt2, a2c2):
    n = raw.shape[1]

    def body(raw_ref, cnt_ref, a_ref, xe_ref, ws_ref):
        i = pl.program_id(0)
        a2 = a_ref[:, 0:1]
        c2 = a_ref[:, 1:2]
        xe = raw_ref[...] * a2 + c2
        xe_ref[...] = xe
        ct = cnt_ref[0:1]
        cs = cnt_ref[1:2]
        xe2 = xe * xe
        blk = jnp.concatenate(
            [jnp.sum(cs * xe, axis=1, keepdims=True),
             jnp.sum(cs * xe2, axis=1, keepdims=True),
             jnp.sum(ct * xe, axis=1, keepdims=True),
             jnp.sum(ct * xe2, axis=1, keepdims=True)], axis=1)

        @pl.when(i == 0)
        def _():
            ws_ref[...] = jnp.zeros_like(ws_ref)

        ws_ref[...] += blk

    return pl.pallas_call(
        body,
        grid=(n // BN,),
        in_specs=[
            pl.BlockSpec((2, BN), lambda i: (0, i)),
            pl.BlockSpec((2, BN), lambda i: (0, i)),
            _full((2, 2)),
        ],
        out_specs=[
            pl.BlockSpec((2, BN), lambda i: (0, i)),
            _full((2, 4)),
        ],
        out_shape=[
            jax.ShapeDtypeStruct((2, n), F32),
            jax.ShapeDtypeStruct((2, 4), F32),
        ],
        compiler_params=_params_tc(),
    )(raw, cnt2, a2c2)


def _dec_ep1(xg2, Wd1, bd1, Wd2, bd2, Wd3, bd3, Wd4, bd4, W1s, W1t, b1f):
    e = xg2.shape[1]

    def body(g_ref, Wd1_r, bd1_r, Wd2_r, bd2_r, Wd3_r, bd3_r, Wd4_r, bd4_r,
             W1s_r, W1t_r, b1f_r, msg_ref, hs_ref):
        i = pl.program_id(0)
        g = g_ref[...]
        xi = g[0:2]
        xj = g[2:4]
        d = xj - xi
        W = Wd1_r[...]
        h = jnp.maximum(_dgf(W[0:2], xi) + _dgf(W[2:4], d) + bd1_r[...], 0.0)
        h = jnp.maximum(_dgf(Wd2_r[...], h) + bd2_r[...], 0.0)
        h = jnp.maximum(_dgf(Wd3_r[...], h) + bd3_r[...], 0.0)
        msg_ref[...] = _dgf(Wd4_r[...], h) + bd4_r[...]

        he = jnp.maximum(
            _dgf(W1s_r[...], xj) + _dgf(W1t_r[...], xi) + b1f_r[...], 0.0)
        s = jnp.sum(he, axis=1, keepdims=True)
        q = jnp.sum(he * he, axis=1, keepdims=True)
        blk = jnp.concatenate([s, q], axis=1)

        @pl.when(i == 0)
        def _():
            hs_ref[...] = jnp.zeros_like(hs_ref)

        hs_ref[...] += blk

    return pl.pallas_call(
        body,
        grid=(e // BE,),
        in_specs=[
            pl.BlockSpec((4, BE), lambda i: (0, i)),
            _full(Wd1.shape), _full(bd1.shape), _full(Wd2.shape),
            _full(bd2.shape), _full(Wd3.shape), _full(bd3.shape),
            _full(Wd4.shape), _full(bd4.shape), _full(W1s.shape),
            _full(W1t.shape), _full(b1f.shape),
        ],
        out_specs=[
            pl.BlockSpec((3, BE), lambda i: (0, i)),
            _full((32, 2)),
        ],
        out_shape=[
            jax.ShapeDtypeStruct((3, e), F32),
            jax.ShapeDtypeStruct((32, 2), F32),
        ],
        compiler_params=_params_tc(),
    )(xg2, Wd1, bd1, Wd2, bd2, Wd3, bd3, Wd4, bd4, W1s, W1t, b1f)


def _ep2_stats(xg2, W1s, W1t, b1f, a4, c4, W2, b2):
    e = xg2.shape[1]

    def body(g_ref, W1s_r, W1t_r, b1f_r, a4_r, c4_r, W2_r, b2_r, rs_ref):
        i = pl.program_id(0)
        g = g_ref[...]
        xi = g[0:2]
        xj = g[2:4]
        he = jnp.maximum(
            _dgf(W1s_r[...], xj) + _dgf(W1t_r[...], xi) + b1f_r[...], 0.0)
        hb = he * a4_r[...] + c4_r[...]
        hr = jnp.maximum(_dgf(W2_r[...], hb) + b2_r[...], 0.0)
        s = jnp.sum(hr, axis=1, keepdims=True)
        q = jnp.sum(hr * hr, axis=1, keepdims=True)
        blk = jnp.concatenate([s, q], axis=1)

        @pl.when(i == 0)
        def _():
            rs_ref[...] = jnp.zeros_like(rs_ref)

        rs_ref[...] += blk

    return pl.pallas_call(
        body,
        grid=(e // BE,),
        in_specs=[
            pl.BlockSpec((4, BE), lambda i: (0, i)),
            _full(W1s.shape), _full(W1t.shape), _full(b1f.shape),
            _full(a4.shape), _full(c4.shape), _full(W2.shape),
            _full(b2.shape),
        ],
        out_specs=_full((32, 2)),
        out_shape=jax.ShapeDtypeStruct((32, 2), F32),
        compiler_params=_params_tc(),
    )(xg2, W1s, W1t, b1f, a4, c4, W2, b2)


def _ep_final(xg2, W1s, W1t, b1f, a4, c4, W2, b2, a5, c5, W4, b4):
    e = xg2.shape[1]

    def body(g_ref, W1s_r, W1t_r, b1f_r, a4_r, c4_r, W2_r, b2_r, a5_r, c5_r,
             W4_r, b4_r, out_ref):
        g = g_ref[...]
        xi = g[0:2]
        xj = g[2:4]
        he = jnp.maximum(
            _dgf(W1s_r[...], xj) + _dgf(W1t_r[...], xi) + b1f_r[...], 0.0)
        hb = he * a4_r[...] + c4_r[...]
        hr = jnp.maximum(_dgf(W2_r[...], hb) + b2_r[...], 0.0)
        hh = hb + hr * a5_r[...] + c5_r[...]
        out_ref[...] = lax.dot_general(
            hh, W4_r[...], (((0,), (0,)), ((), ())),
            preferred_element_type=F32) + b4_r[...]

    return pl.pallas_call(
        body,
        grid=(e // BE,),
        in_specs=[
            pl.BlockSpec((4, BE), lambda i: (0, i)),
            _full(W1s.shape), _full(W1t.shape), _full(b1f.shape),
            _full(a4.shape), _full(c4.shape), _full(W2.shape),
            _full(b2.shape), _full(a5.shape), _full(c5.shape),
            _full(W4.shape), _full(b4.shape),
        ],
        out_specs=pl.BlockSpec((BE, 4), lambda i: (i, 0)),
        out_shape=jax.ShapeDtypeStruct((e, 4), F32),
        compiler_params=_params_tc(),
    )(xg2, W1s, W1t, b1f, a4, c4, W2, b2, a5, c5, W4, b4)


def _recon_finalize(part2, cnt2):
    n = part2.shape[2]

    def body(p_ref, cnt_ref, rec_ref):
        p = jnp.sum(p_ref[...], axis=1)
        safe = jnp.maximum(cnt_ref[0:1], 1.0)
        rec_ref[...] = p / safe

    return pl.pallas_call(
        body,
        grid=(n // BN,),
        in_specs=[
            pl.BlockSpec((3, NW, BN), lambda i: (0, 0, i)),
            pl.BlockSpec((2, BN), lambda i: (0, i)),
        ],
        out_specs=pl.BlockSpec((3, BN), lambda i: (0, i)),
        out_shape=jax.ShapeDtypeStruct((3, n), F32),
        compiler_params=_params_tc(),
    )(part2, cnt2)


def _affine(g, b, s, q, count):
    m = s / count
    v = q / count - m * m
    a = g / jnp.sqrt(v + EPS)
    return a, b - m * a


def kernel(x, edge_index, edge_attr, params):
    n = x.shape[0]
    e = edge_index.shape[1]
    npad = -(-n // BN) * BN
    src = edge_index[0].astype(jnp.int32)
    tgt = edge_index[1].astype(jnp.int32)
    xT = jnp.pad(x.T, ((0, 0), (0, npad - n)))

    st1 = _colstats(xT)
    a1, c1 = _affine(params['bn1g'], params['bn1b'], st1[:, 0], st1[:, 1], n)
    We0, be0 = params['enc'][0]
    Wx = We0[:3] * a1[:, None]
    Wea = We0[3:]
    b0 = (be0 + c1 @ We0[:3]).reshape(32, 1)

    xg = _sc_gather_x(xT, tgt)
    (W2e, b2e), (W3e, b3e), (W4e, b4e) = params['enc'][1:]
    msgT = _enc_mlp(xg, edge_attr, Wx, Wea, b0, W2e, b2e.reshape(-1, 1),
                    W3e, b3e.reshape(-1, 1), W4e, b4e.reshape(-1, 1))

    part = _sc_scatter4(tgt, src, msgT, npad)
    raw, cnt2, rst = _node_finalize_a(part)

    a2, c2 = _affine(params['bn2g'], params['bn2b'], rst[:, 0], rst[:, 1], n)
    a2c2 = jnp.stack([a2, c2], axis=1)
    xeT, ws = _node_finalize_b(raw, cnt2, a2c2)

    S_s, Q_s, S_t, Q_t = ws[:, 0], ws[:, 1], ws[:, 2], ws[:, 3]
    a3, c3 = _affine(params['epbn1g'], params['epbn1b'],
                     jnp.concatenate([S_s, S_t]),
                     jnp.concatenate([Q_s, Q_t]), e)
    W1, b1 = params['ep_fc1']
    W1s = W1[:2] * a3[:2, None]
    W1t = W1[2:] * a3[2:, None]
    b1f = (b1 + c3 @ W1).reshape(32, 1)

    xg2 = _sc_gather_xe(xeT, tgt, src)
    (Wd1, bd1), (Wd2, bd2), (Wd3, bd3), (Wd4, bd4) = params['dec']
    msg2T, hs = _dec_ep1(
        xg2, Wd1, bd1.reshape(-1, 1), Wd2, bd2.reshape(-1, 1),
        Wd3, bd3.reshape(-1, 1), Wd4, bd4.reshape(-1, 1), W1s, W1t, b1f)

    part2 = _sc_scatter3(tgt, msg2T, npad)
    recT = _recon_finalize(part2, cnt2)
    x_recon = recT[:, :n].T

    a4, c4 = _affine(params['epbn2g'], params['epbn2b'], hs[:, 0], hs[:, 1], e)
    a4 = a4.reshape(32, 1)
    c4 = c4.reshape(32, 1)
    W2, b2 = params['ep_fc2']
    rs = _ep2_stats(xg2, W1s, W1t, b1f, a4, c4, W2, b2.reshape(-1, 1))

    a5, c5 = _affine(params['epbn3g'], params['epbn3b'], rs[:, 0], rs[:, 1], e)
    W4, b4 = params['ep_fc4']
    pred = _ep_final(xg2, W1s, W1t, b1f, a4, c4, W2, b2.reshape(-1, 1),
                     a5.reshape(32, 1), c5.reshape(32, 1), W4,
                     b4.reshape(1, -1))
    return (x_recon, pred)


# speedup vs baseline: 9.7535x; 1.5459x over previous
import functools

import jax
import jax.numpy as jnp
from jax import lax
from jax.experimental import pallas as pl
from jax.experimental.pallas import tpu as pltpu
from jax.experimental.pallas import tpu_sc as plsc

F32 = jnp.float32
EPS = 1e-5

NW = 32
SB = 10000
BE = 8192
BN = 4096

_MESH = None


def _mesh():
    global _MESH
    if _MESH is None:
        _MESH = plsc.VectorSubcoreMesh(
            core_axis_name="c", subcore_axis_name="s", num_cores=2,
            num_subcores=16)
    return _MESH


_SC_PARAMS = pltpu.CompilerParams(needs_layout_passes=False)


def _wid():
    return lax.axis_index("s") * 2 + lax.axis_index("c")


def _worker_blocks(wid, nblk, fn):
    nb = nblk // NW

    def body(k, _):
        fn((wid * nb + k) * SB)
        return 0

    lax.fori_loop(0, nb, body, 0)


def _gather_block(table_v, idx_v, out_v):
    def body(i, _):
        base = i * 80
        for u in range(5):
            o = base + u * 16
            idx = idx_v[pl.ds(o, 16)]
            out_v[pl.ds(o, 16)] = plsc.load_gather(table_v, [idx])
        return 0

    lax.fori_loop(0, SB // 80, body, 0)


def _scatter_block(acc_v, idx_v, val_v):
    def body(i, _):
        base = i * 80
        for u in range(5):
            o = base + u * 16
            idx = idx_v[pl.ds(o, 16)]
            v = val_v[pl.ds(o, 16)]
            plsc.addupdate_scatter(acc_v, [idx], v)
        return 0

    lax.fori_loop(0, SB // 80, body, 0)


def _fill(ref, n, value):
    vec = jnp.full((16,), value, F32)

    def body(i, _):
        base = i * 80
        for u in range(5):
            ref[pl.ds(base + u * 16, 16)] = vec
        return 0

    lax.fori_loop(0, n // 80, body, 0)


def _sc_gather_x(xt_flat, tgt):
    n = xt_flat.shape[0] // 3
    e = tgt.shape[0]
    nblk = e // SB
    shp = jax.ShapeDtypeStruct((e,), F32)

    @functools.partial(
        pl.kernel,
        out_type=[shp, shp, shp],
        mesh=_mesh(),
        scratch_types=[
            pltpu.VMEM((n,), F32),
            pltpu.VMEM((SB,), jnp.int32),
            pltpu.VMEM((SB,), F32),
        ],
        compiler_params=_SC_PARAMS,
    )
    def k(xt_h, tgt_h, o0_h, o1_h, o2_h, table_v, idx_v, out_v):
        wid = _wid()
        for j, o_h in enumerate((o0_h, o1_h, o2_h)):
            pltpu.sync_copy(xt_h.at[pl.ds(j * n, n)], table_v)

            def do(base, o_h=o_h):
                pltpu.sync_copy(tgt_h.at[pl.ds(base, SB)], idx_v)
                _gather_block(table_v, idx_v, out_v)
                pltpu.sync_copy(out_v, o_h.at[pl.ds(base, SB)])

            _worker_blocks(wid, nblk, do)

    return k(xt_flat, tgt)


def _sc_gather_xe(xe0, xe1, tgt, src):
    e = tgt.shape[0]
    n = xe0.shape[0]
    nblk = e // SB
    shp = jax.ShapeDtypeStruct((e,), F32)

    @functools.partial(
        pl.kernel,
        out_type=[shp, shp, shp, shp],
        mesh=_mesh(),
        scratch_types=[
            pltpu.VMEM((n,), F32),
            pltpu.VMEM((SB,), jnp.int32),
            pltpu.VMEM((SB,), F32),
        ],
        compiler_params=_SC_PARAMS,
    )
    def k(xe0_h, xe1_h, tgt_h, src_h, o0_h, o1_h, o2_h, o3_h,
          table_v, idx_v, out_v):
        wid = _wid()
        for tab_h, outs in ((xe0_h, (o0_h, o2_h)), (xe1_h, (o1_h, o3_h))):
            pltpu.sync_copy(tab_h, table_v)
            for o_h, idx_h in zip(outs, (tgt_h, src_h)):

                def do(base, o_h=o_h, idx_h=idx_h):
                    pltpu.sync_copy(idx_h.at[pl.ds(base, SB)], idx_v)
                    _gather_block(table_v, idx_v, out_v)
                    pltpu.sync_copy(out_v, o_h.at[pl.ds(base, SB)])

                _worker_blocks(wid, nblk, do)

    return k(xe0, xe1, tgt, src)


def _sc_scatter4(tgt, src, msg0, msg1, n):
    e = tgt.shape[0]
    nblk = e // SB
    shp = jax.ShapeDtypeStruct((NW * n,), F32)

    @functools.partial(
        pl.kernel,
        out_type=[shp, shp, shp, shp],
        mesh=_mesh(),
        scratch_types=[
            pltpu.VMEM((n,), F32),
            pltpu.VMEM((SB,), jnp.int32),
            pltpu.VMEM((SB,), F32),
        ],
        compiler_params=_SC_PARAMS,
    )
    def k(tgt_h, src_h, m0_h, m1_h, o0_h, o1_h, o2_h, o3_h,
          acc_v, idx_v, val_v):
        wid = _wid()
        for col, (o_h, idx_h, val_h) in enumerate((
                (o0_h, tgt_h, m0_h), (o1_h, tgt_h, m1_h),
                (o2_h, tgt_h, None), (o3_h, src_h, None))):
            _fill(acc_v, n, 0.0)
            if val_h is None:
                _fill(val_v, SB, 1.0)

            def do(base, idx_h=idx_h, val_h=val_h):
                pltpu.sync_copy(idx_h.at[pl.ds(base, SB)], idx_v)
                if val_h is not None:
                    pltpu.sync_copy(val_h.at[pl.ds(base, SB)], val_v)
                _scatter_block(acc_v, idx_v, val_v)

            _worker_blocks(wid, nblk, do)
            pltpu.sync_copy(acc_v, o_h.at[pl.ds(wid * n, n)])

    return k(tgt, src, msg0, msg1)


def _sc_scatter3(tgt, m0, m1, m2, n):
    e = tgt.shape[0]
    nblk = e // SB
    shp = jax.ShapeDtypeStruct((NW * n,), F32)

    @functools.partial(
        pl.kernel,
        out_type=[shp, shp, shp],
        mesh=_mesh(),
        scratch_types=[
            pltpu.VMEM((n,), F32),
            pltpu.VMEM((SB,), jnp.int32),
            pltpu.VMEM((SB,), F32),
        ],
        compiler_params=_SC_PARAMS,
    )
    def k(tgt_h, m0_h, m1_h, m2_h, o0_h, o1_h, o2_h, acc_v, idx_v, val_v):
        wid = _wid()
        for o_h, val_h in ((o0_h, m0_h), (o1_h, m1_h), (o2_h, m2_h)):
            _fill(acc_v, n, 0.0)

            def do(base, val_h=val_h):
                pltpu.sync_copy(tgt_h.at[pl.ds(base, SB)], idx_v)
                pltpu.sync_copy(val_h.at[pl.ds(base, SB)], val_v)
                _scatter_block(acc_v, idx_v, val_v)

            _worker_blocks(wid, nblk, do)
            pltpu.sync_copy(acc_v, o_h.at[pl.ds(wid * n, n)])

    return k(tgt, m0, m1, m2)


def _dgf(a, x):
    return lax.dot_general(a, x, (((0,), (0,)), ((), ())),
                           preferred_element_type=F32)


def _params_tc(dimsem=("arbitrary",)):
    return pltpu.CompilerParams(dimension_semantics=dimsem)


def _full(shape):
    return pl.BlockSpec(shape, lambda *a: (0,) * len(shape))


def _row(r):
    return r[...].reshape(1, BE)


def _colstats(xT):
    f, n = xT.shape

    def body(x_ref, st_ref):
        i = pl.program_id(0)
        xb = x_ref[...]
        s = jnp.sum(xb, axis=1, keepdims=True)
        q = jnp.sum(xb * xb, axis=1, keepdims=True)
        blk = jnp.concatenate([s, q], axis=1)

        @pl.when(i == 0)
        def _():
            st_ref[...] = jnp.zeros_like(st_ref)

        st_ref[...] += blk

    return pl.pallas_call(
        body,
        grid=(n // BN,),
        in_specs=[pl.BlockSpec((f, BN), lambda i: (0, i))],
        out_specs=_full((f, 2)),
        out_shape=jax.ShapeDtypeStruct((f, 2), F32),
        compiler_params=_params_tc(),
    )(xT)


def _enc_mlp(xg0, xg1, xg2, eaT, a1c1, Wx, Wea, b0, W2, b2, W3, b3, W4,
             b4):
    e = eaT.shape[1]

    def body(x0_r, x1_r, x2_r, ea_ref, a1_r, Wx_r, Wea_r, b0_r, W2_r, b2_r,
             W3_r, b3_r, W4_r, b4_r, o0_ref, o1_ref):
        xg = jnp.concatenate([_row(x0_r), _row(x1_r), _row(x2_r)], axis=0)
        a1 = a1_r[...]
        xn = xg * a1[:, 0:1] + a1[:, 1:2]
        h = _dgf(Wx_r[...], xn)
        h = h + _dgf(Wea_r[...], ea_ref[...])
        h = jnp.maximum(h + b0_r[...], 0.0)
        h = jnp.maximum(_dgf(W2_r[...], h) + b2_r[...], 0.0)
        h = jnp.maximum(_dgf(W3_r[...], h) + b3_r[...], 0.0)
        h = jnp.maximum(_dgf(W4_r[...], h) + b4_r[...], 0.0)
        o0_ref[...] = h[0]
        o1_ref[...] = h[1]

    eb = pl.BlockSpec((BE,), lambda i: (i,))
    shp = jax.ShapeDtypeStruct((e,), F32)
    return pl.pallas_call(
        body,
        grid=(-(-e // BE),),
        in_specs=[
            eb, eb, eb,
            pl.BlockSpec((4, BE), lambda i: (0, i)),
            _full((3, 2)), _full(Wx.shape), _full(Wea.shape), _full(b0.shape),
            _full(W2.shape), _full(b2.shape), _full(W3.shape),
            _full(b3.shape), _full(W4.shape), _full(b4.shape),
        ],
        out_specs=[eb, eb],
        out_shape=[shp, shp],
        compiler_params=_params_tc(),
    )(xg0, xg1, xg2, eaT, a1c1, Wx, Wea, b0, W2, b2, W3, b3, W4, b4)


def _node_finalize_a(p0, p1, p2, p3, n):
    jb = n // BN

    def body(p0_r, p1_r, p2_r, p3_r, r0_ref, r1_ref, ct_ref, cs_ref,
             st_ref):
        j = pl.program_id(0)
        w = pl.program_id(1)

        @pl.when(w == 0)
        def _():
            r0_ref[...] = jnp.zeros_like(r0_ref)
            r1_ref[...] = jnp.zeros_like(r1_ref)
            ct_ref[...] = jnp.zeros_like(ct_ref)
            cs_ref[...] = jnp.zeros_like(cs_ref)

        r0_ref[...] += p0_r[...]
        r1_ref[...] += p1_r[...]
        ct_ref[...] += p2_r[...]
        cs_ref[...] += p3_r[...]

        @pl.when(w == NW - 1)
        def _():
            safe = jnp.maximum(ct_ref[...], 1.0)
            raw0 = r0_ref[...] / safe
            raw1 = r1_ref[...] / safe
            r0_ref[...] = raw0
            r1_ref[...] = raw1
            a = raw0.reshape(1, BN)
            b = raw1.reshape(1, BN)
            blk = jnp.concatenate(
                [jnp.sum(a, axis=1, keepdims=True),
                 jnp.sum(a * a, axis=1, keepdims=True),
                 jnp.sum(b, axis=1, keepdims=True),
                 jnp.sum(b * b, axis=1, keepdims=True)], axis=1)

            @pl.when(j == 0)
            def _():
                st_ref[...] = jnp.zeros_like(st_ref)

            st_ref[...] += blk

    pb = pl.BlockSpec((BN,), lambda j, w: (w * jb + j,))
    nb = pl.BlockSpec((BN,), lambda j, w: (j,))
    shp = jax.ShapeDtypeStruct((n,), F32)
    return pl.pallas_call(
        body,
        grid=(jb, NW),
        in_specs=[pb, pb, pb, pb],
        out_specs=[nb, nb, nb, nb, _full((1, 4))],
        out_shape=[shp, shp, shp, shp, jax.ShapeDtypeStruct((1, 4), F32)],
        compiler_params=_params_tc(("arbitrary", "arbitrary")),
    )(p0, p1, p2, p3)


def _node_finalize_b(raw0, raw1, ct, cs, a2c2):
    n = raw0.shape[0]

    def body(r0_r, r1_r, ct_r, cs_r, a_ref, xe0_ref, xe1_ref, ws_ref):
        j = pl.program_id(0)
        a = a_ref[...]
        xe0 = r0_r[...] * a[0, 0] + a[0, 1]
        xe1 = r1_r[...] * a[0, 2] + a[0, 3]
        xe0_ref[...] = xe0
        xe1_ref[...] = xe1
        ct = ct_r[...].reshape(1, BN)
        cs = cs_r[...].reshape(1, BN)
        e0 = xe0.reshape(1, BN)
        e1 = xe1.reshape(1, BN)
        cols = [cs * e0, cs * e0 * e0, cs * e1, cs * e1 * e1,
                ct * e0, ct * e0 * e0, ct * e1, ct * e1 * e1]
        blk = jnp.concatenate(
            [jnp.sum(c, axis=1, keepdims=True) for c in cols], axis=1)

        @pl.when(j == 0)
        def _():
            ws_ref[...] = jnp.zeros_like(ws_ref)

        ws_ref[...] += blk

    nb = pl.BlockSpec((BN,), lambda j: (j,))
    shp = jax.ShapeDtypeStruct((n,), F32)
    return pl.pallas_call(
        body,
        grid=(n // BN,),
        in_specs=[nb, nb, nb, nb, _full((1, 4))],
        out_specs=[nb, nb, _full((1, 8))],
        out_shape=[shp, shp, jax.ShapeDtypeStruct((1, 8), F32)],
        compiler_params=_params_tc(),
    )(raw0, raw1, ct, cs, a2c2)


def _dec_ep1(g0, g1, g2, g3, Wd1, bd1, Wd2, bd2, Wd3, bd3, Wd4, bd4,
             a3c3, W1s, W1t, b1f):
    e = g0.shape[0]

    def body(g0_r, g1_r, g2_r, g3_r, Wd1_r, bd1_r, Wd2_r, bd2_r, Wd3_r,
             bd3_r, Wd4_r, bd4_r, a3_r, W1s_r, W1t_r, b1f_r,
             m0_ref, m1_ref, m2_ref, hs_ref):
        i = pl.program_id(0)
        xi = jnp.concatenate([_row(g0_r), _row(g1_r)], axis=0)
        xj = jnp.concatenate([_row(g2_r), _row(g3_r)], axis=0)
        d = xj - xi
        W = Wd1_r[...]
        h = jnp.maximum(_dgf(W[0:2], xi) + _dgf(W[2:4], d) + bd1_r[...], 0.0)
        h = jnp.maximum(_dgf(Wd2_r[...], h) + bd2_r[...], 0.0)
        h = jnp.maximum(_dgf(Wd3_r[...], h) + bd3_r[...], 0.0)
        m = _dgf(Wd4_r[...], h) + bd4_r[...]
        m0_ref[...] = m[0]
        m1_ref[...] = m[1]
        m2_ref[...] = m[2]

        a3 = a3_r[...]
        xjn = xj * a3[0:2, 0:1] + a3[0:2, 1:2]
        xin = xi * a3[2:4, 0:1] + a3[2:4, 1:2]
        he = jnp.maximum(
            _dgf(W1s_r[...], xjn) + _dgf(W1t_r[...], xin) + b1f_r[...], 0.0)
        col = lax.broadcasted_iota(jnp.int32, (1, BE), 1) + i * BE
        hm = jnp.where(col < e, he, 0.0)
        s = jnp.sum(hm, axis=1, keepdims=True)
        q = jnp.sum(hm * hm, axis=1, keepdims=True)
        blk = jnp.concatenate([s, q], axis=1)

        @pl.when(i == 0)
        def _():
            hs_ref[...] = jnp.zeros_like(hs_ref)

        hs_ref[...] += blk

    eb = pl.BlockSpec((BE,), lambda i: (i,))
    shp = jax.ShapeDtypeStruct((e,), F32)
    return pl.pallas_call(
        body,
        grid=(-(-e // BE),),
        in_specs=[
            eb, eb, eb, eb,
            _full(Wd1.shape), _full(bd1.shape), _full(Wd2.shape),
            _full(bd2.shape), _full(Wd3.shape), _full(bd3.shape),
            _full(Wd4.shape), _full(bd4.shape), _full((4, 2)),
            _full(W1s.shape), _full(W1t.shape), _full(b1f.shape),
        ],
        out_specs=[eb, eb, eb, _full((32, 2))],
        out_shape=[shp, shp, shp, jax.ShapeDtypeStruct((32, 2), F32)],
        compiler_params=_params_tc(),
    )(g0, g1, g2, g3, Wd1, bd1, Wd2, bd2, Wd3, bd3, Wd4, bd4, a3c3, W1s,
      W1t, b1f)


def _ep2_stats(g0, g1, g2, g3, a3c3, W1s, W1t, b1f, a4, c4, W2, b2):
    e = g0.shape[0]

    def body(g0_r, g1_r, g2_r, g3_r, a3_r, W1s_r, W1t_r, b1f_r, a4_r, c4_r,
             W2_r, b2_r, rs_ref):
        i = pl.program_id(0)
        xi = jnp.concatenate([_row(g0_r), _row(g1_r)], axis=0)
        xj = jnp.concatenate([_row(g2_r), _row(g3_r)], axis=0)
        a3 = a3_r[...]
        xjn = xj * a3[0:2, 0:1] + a3[0:2, 1:2]
        xin = xi * a3[2:4, 0:1] + a3[2:4, 1:2]
        he = jnp.maximum(
            _dgf(W1s_r[...], xjn) + _dgf(W1t_r[...], xin) + b1f_r[...], 0.0)
        hb = he * a4_r[...] + c4_r[...]
        hr = jnp.maximum(_dgf(W2_r[...], hb) + b2_r[...], 0.0)
        col = lax.broadcasted_iota(jnp.int32, (1, BE), 1) + i * BE
        hm = jnp.where(col < e, hr, 0.0)
        s = jnp.sum(hm, axis=1, keepdims=True)
        q = jnp.sum(hm * hm, axis=1, keepdims=True)
        blk = jnp.concatenate([s, q], axis=1)

        @pl.when(i == 0)
        def _():
            rs_ref[...] = jnp.zeros_like(rs_ref)

        rs_ref[...] += blk

    eb = pl.BlockSpec((BE,), lambda i: (i,))
    return pl.pallas_call(
        body,
        grid=(-(-e // BE),),
        in_specs=[
            eb, eb, eb, eb, _full((4, 2)),
            _full(W1s.shape), _full(W1t.shape), _full(b1f.shape),
            _full(a4.shape), _full(c4.shape), _full(W2.shape),
            _full(b2.shape),
        ],
        out_specs=_full((32, 2)),
        out_shape=jax.ShapeDtypeStruct((32, 2), F32),
        compiler_params=_params_tc(),
    )(g0, g1, g2, g3, a3c3, W1s, W1t, b1f, a4, c4, W2, b2)


def _ep_final(g0, g1, g2, g3, a3c3, W1s, W1t, b1f, a4, c4, W2, b2, a5, c5,
              W4, b4):
    e = g0.shape[0]

    def body(g0_r, g1_r, g2_r, g3_r, a3_r, W1s_r, W1t_r, b1f_r, a4_r, c4_r,
             W2_r, b2_r, a5_r, c5_r, W4_r, b4_r, out_ref):
        xi = jnp.concatenate([_row(g0_r), _row(g1_r)], axis=0)
        xj = jnp.concatenate([_row(g2_r), _row(g3_r)], axis=0)
        a3 = a3_r[...]
        xjn = xj * a3[0:2, 0:1] + a3[0:2, 1:2]
        xin = xi * a3[2:4, 0:1] + a3[2:4, 1:2]
        he = jnp.maximum(
            _dgf(W1s_r[...], xjn) + _dgf(W1t_r[...], xin) + b1f_r[...], 0.0)
        hb = he * a4_r[...] + c4_r[...]
        hr = jnp.maximum(_dgf(W2_r[...], hb) + b2_r[...], 0.0)
        hh = hb + hr * a5_r[...] + c5_r[...]
        out_ref[...] = lax.dot_general(
            hh, W4_r[...], (((0,), (0,)), ((), ())),
            preferred_element_type=F32) + b4_r[...]

    eb = pl.BlockSpec((BE,), lambda i: (i,))
    return pl.pallas_call(
        body,
        grid=(-(-e // BE),),
        in_specs=[
            eb, eb, eb, eb, _full((4, 2)),
            _full(W1s.shape), _full(W1t.shape), _full(b1f.shape),
            _full(a4.shape), _full(c4.shape), _full(W2.shape),
            _full(b2.shape), _full(a5.shape), _full(c5.shape),
            _full(W4.shape), _full(b4.shape),
        ],
        out_specs=pl.BlockSpec((BE, 4), lambda i: (i, 0)),
        out_shape=jax.ShapeDtypeStruct((e, 4), F32),
        compiler_params=_params_tc(),
    )(g0, g1, g2, g3, a3c3, W1s, W1t, b1f, a4, c4, W2, b2, a5, c5, W4, b4)


def _recon_finalize(q0, q1, q2, ct, n):
    jb = n // BN

    def body(q0_r, q1_r, q2_r, ct_r, r0_ref, r1_ref, r2_ref):
        w = pl.program_id(1)

        @pl.when(w == 0)
        def _():
            r0_ref[...] = jnp.zeros_like(r0_ref)
            r1_ref[...] = jnp.zeros_like(r1_ref)
            r2_ref[...] = jnp.zeros_like(r2_ref)

        r0_ref[...] += q0_r[...]
        r1_ref[...] += q1_r[...]
        r2_ref[...] += q2_r[...]

        @pl.when(w == NW - 1)
        def _():
            inv = 1.0 / jnp.maximum(ct_r[...], 1.0)
            r0_ref[...] *= inv
            r1_ref[...] *= inv
            r2_ref[...] *= inv

    pb = pl.BlockSpec((BN,), lambda j, w: (w * jb + j,))
    nb = pl.BlockSpec((BN,), lambda j, w: (j,))
    shp = jax.ShapeDtypeStruct((n,), F32)
    return pl.pallas_call(
        body,
        grid=(jb, NW),
        in_specs=[pb, pb, pb, nb],
        out_specs=[nb, nb, nb],
        out_shape=[shp, shp, shp],
        compiler_params=_params_tc(("arbitrary", "arbitrary")),
    )(q0, q1, q2, ct)


def _affine(g, b, s, q, count):
    m = s / count
    v = q / count - m * m
    a = g / jnp.sqrt(v + EPS)
    return a, b - m * a


def kernel(x, edge_index, edge_attr, params):
    n = x.shape[0]
    e = edge_index.shape[1]
    npad = -(-n // BN) * BN
    src = edge_index[0].astype(jnp.int32)
    tgt = edge_index[1].astype(jnp.int32)
    xT = jnp.pad(x.T, ((0, 0), (0, npad - n)))

    st1 = _colstats(xT)
    a1, c1 = _affine(params['bn1g'], params['bn1b'], st1[:, 0], st1[:, 1], n)
    We0, be0 = params['enc'][0]
    Wx = We0[:3]
    Wea = We0[3:]
    b0 = be0.reshape(32, 1)
    a1c1 = jnp.stack([a1, c1], axis=1)

    xg0, xg1, xg2 = _sc_gather_x(xT.reshape(-1), tgt)
    (W2e, b2e), (W3e, b3e), (W4e, b4e) = params['enc'][1:]
    msg0, msg1 = _enc_mlp(xg0, xg1, xg2, edge_attr.T, a1c1, Wx, Wea, b0,
                          W2e, b2e.reshape(-1, 1), W3e, b3e.reshape(-1, 1),
                          W4e, b4e.reshape(-1, 1))

    p0, p1, p2, p3 = _sc_scatter4(tgt, src, msg0, msg1, npad)
    raw0, raw1, cnt_t, cnt_s, rst = _node_finalize_a(p0, p1, p2, p3, npad)

    a2, c2 = _affine(params['bn2g'], params['bn2b'],
                     jnp.stack([rst[0, 0], rst[0, 2]]),
                     jnp.stack([rst[0, 1], rst[0, 3]]), n)
    a2c2 = jnp.stack([a2[0], c2[0], a2[1], c2[1]]).reshape(1, 4)
    xe0, xe1, ws = _node_finalize_b(raw0, raw1, cnt_t, cnt_s, a2c2)

    a3, c3 = _affine(params['epbn1g'], params['epbn1b'],
                     jnp.stack([ws[0, 0], ws[0, 2], ws[0, 4], ws[0, 6]]),
                     jnp.stack([ws[0, 1], ws[0, 3], ws[0, 5], ws[0, 7]]), e)
    W1, b1 = params['ep_fc1']
    W1s = W1[:2]
    W1t = W1[2:]
    b1f = b1.reshape(32, 1)
    a3c3 = jnp.stack([a3, c3], axis=1)

    g0, g1, g2, g3 = _sc_gather_xe(xe0, xe1, tgt, src)
    (Wd1, bd1), (Wd2, bd2), (Wd3, bd3), (Wd4, bd4) = params['dec']
    m0, m1, m2, hs = _dec_ep1(
        g0, g1, g2, g3, Wd1, bd1.reshape(-1, 1), Wd2, bd2.reshape(-1, 1),
        Wd3, bd3.reshape(-1, 1), Wd4, bd4.reshape(-1, 1), a3c3, W1s, W1t,
        b1f)

    q0, q1, q2 = _sc_scatter3(tgt, m0, m1, m2, npad)
    r0, r1, r2 = _recon_finalize(q0, q1, q2, cnt_t, npad)
    x_recon = jnp.stack([r0[:n], r1[:n], r2[:n]], axis=1)

    a4, c4 = _affine(params['epbn2g'], params['epbn2b'], hs[:, 0], hs[:, 1], e)
    a4 = a4.reshape(32, 1)
    c4 = c4.reshape(32, 1)
    W2, b2 = params['ep_fc2']
    rs = _ep2_stats(g0, g1, g2, g3, a3c3, W1s, W1t, b1f, a4, c4, W2,
                    b2.reshape(-1, 1))

    a5, c5 = _affine(params['epbn3g'], params['epbn3b'], rs[:, 0], rs[:, 1], e)
    W4, b4 = params['ep_fc4']
    pred = _ep_final(g0, g1, g2, g3, a3c3, W1s, W1t, b1f, a4, c4, W2,
                     b2.reshape(-1, 1), a5.reshape(32, 1), c5.reshape(32, 1),
                     W4, b4.reshape(1, -1))
    return (x_recon, pred)
